# baseline (speedup 1.0000x reference)
import functools

import jax
import jax.numpy as jnp
from jax import lax
from jax.experimental import pallas as pl
from jax.experimental.pallas import tpu as pltpu
from jax.experimental.pallas import tpu_sc as plsc

NC = 2
NS = 16
L = 16
NW = NC * NS

GW = 128
CHUNK_GW = 8


def kernel(x, table, scale):
    B_, S_ = x.shape
    V, D = table.shape
    B = B_ * S_

    idx2d = x.reshape(B // GW, GW).astype(jnp.int32)
    scale_vec = jnp.broadcast_to(scale.astype(jnp.float32), (L,))

    n_gw_per_w = (B // GW) // NW
    n_chunks = n_gw_per_w // CHUNK_GW
    rows_per_chunk = CHUNK_GW * GW

    mesh = plsc.VectorSubcoreMesh(core_axis_name="c", subcore_axis_name="s")

    @functools.partial(
        pl.kernel,
        out_type=jax.ShapeDtypeStruct((B, D), jnp.float32),
        mesh=mesh,
        scratch_types=[
            pltpu.VMEM((CHUNK_GW, GW), jnp.int32),
            pltpu.VMEM((rows_per_chunk, D), jnp.float32),
            pltpu.VMEM((L,), jnp.float32),
            pltpu.SemaphoreType.DMA,
        ],
        compiler_params=pltpu.CompilerParams(use_tc_tiling_on_sc=False),
    )
    def emb_kernel(idx_hbm, table_hbm, scale_hbm, out_hbm,
                   idx_v, rows_v, scale_v, sem):
        wid = lax.axis_index("s") * NC + lax.axis_index("c")
        pltpu.sync_copy(scale_hbm, scale_v)
        sv = scale_v[...]

        @pl.loop(0, n_chunks)
        def _(ci):
            gw0 = wid * n_gw_per_w + ci * CHUNK_GW
            pltpu.sync_copy(idx_hbm.at[pl.ds(gw0, CHUNK_GW)], idx_v)
            copies = [
                pltpu.async_copy(table_hbm.at[idx_v.at[j]],
                                 rows_v.at[pl.ds(j * GW, GW)], sem)
                for j in range(CHUNK_GW)
            ]
            for c in copies:
                c.wait()

            @pl.loop(0, rows_per_chunk)
            def _(r):
                for jj in range(D // L):
                    sl = pl.ds(jj * L, L)
                    rows_v[r, sl] = rows_v[r, sl] * sv

            pltpu.sync_copy(rows_v, out_hbm.at[pl.ds(gw0 * GW, rows_per_chunk)])

    out = emb_kernel(idx2d, table, scale_vec)
    return out.reshape(B_, S_, D)


# speedup vs baseline: 1.0040x; 1.0040x over previous
import functools

import jax
import jax.numpy as jnp
from jax import lax
from jax.experimental import pallas as pl
from jax.experimental.pallas import tpu as pltpu
from jax.experimental.pallas import tpu_sc as plsc

NC = 2
NS = 16
L = 16
NW = NC * NS

NB = 2


def kernel(x, table, scale):
    B, S = x.shape
    V, D = table.shape

    xi = x.astype(jnp.int32)
    scale_vec = jnp.broadcast_to(scale.astype(jnp.float32), (L,))

    b_per_w = B // NW
    n_chunks = b_per_w // NB
    s_hi = (S // 2 + 7) // 8 * 8
    s_lo = S - s_hi

    mesh = plsc.VectorSubcoreMesh(core_axis_name="c", subcore_axis_name="s")

    @functools.partial(
        pl.kernel,
        out_type=jax.ShapeDtypeStruct((B, S, D), jnp.float32),
        mesh=mesh,
        scratch_types=[
            pltpu.VMEM((NB, S), jnp.int32),
            pltpu.VMEM((NB, S, D), jnp.float32),
            pltpu.VMEM((L,), jnp.float32),
            pltpu.SemaphoreType.DMA,
        ],
        compiler_params=pltpu.CompilerParams(use_tc_tiling_on_sc=False),
    )
    def emb_kernel(idx_hbm, table_hbm, scale_hbm, out_hbm,
                   idx_v, rows_v, scale_v, sem):
        wid = lax.axis_index("s") * NC + lax.axis_index("c")
        pltpu.sync_copy(scale_hbm, scale_v)
        sv = scale_v[...]

        @pl.loop(0, n_chunks)
        def _(ci):
            b0 = wid * b_per_w + ci * NB
            pltpu.sync_copy(idx_hbm.at[pl.ds(b0, NB)], idx_v)
            copies = []
            for i in range(NB):
                copies.append(pltpu.async_copy(
                    table_hbm.at[idx_v.at[i, pl.ds(0, s_hi)]],
                    rows_v.at[i, pl.ds(0, s_hi)], sem))
                copies.append(pltpu.async_copy(
                    table_hbm.at[idx_v.at[i, pl.ds(s_hi, s_lo)]],
                    rows_v.at[i, pl.ds(s_hi, s_lo)], sem))
            for c in copies:
                c.wait()

            @pl.loop(0, S)
            def _(r):
                for i in range(NB):
                    for jj in range(D // L):
                        sl = pl.ds(jj * L, L)
                        rows_v[i, r, sl] = rows_v[i, r, sl] * sv

            pltpu.sync_copy(rows_v, out_hbm.at[pl.ds(b0, NB)])

    return emb_kernel(xi, table, scale_vec)
